# Initial kernel scaffold; baseline (speedup 1.0000x reference)
import jax
import jax.numpy as jnp
from jax.experimental import pallas as pl


def kernel(view1, view2, spatial_edge_index, spatial_edge_weight, W1, b1, W2, b2):
    raise NotImplementedError("write your pallas kernel here")


# speedup vs baseline: 3.6129x; 3.6129x over previous
import functools

import jax
import jax.numpy as jnp
from jax import lax
from jax.experimental import pallas as pl
from jax.experimental.pallas import tpu as pltpu
from jax.experimental.pallas import tpu_sc as plsc

T = 8
N = 10000
D = 128
E = 320000
TEMP = 0.1

NCORES = 2
NSUB = 16
HD = D // 4
NH = D // HD
EPT = E // NSUB
CHUNK = 80
NCHUNKS = EPT // CHUNK
NPAD = ((N + 16 * NSUB - 1) // (16 * NSUB)) * (16 * NSUB)
ROWS_MAIN = ((N // NSUB + 7) // 8) * 8
ROWS_LAST = N - (NSUB - 1) * ROWS_MAIN


def _rsqrt16(d):
    i = plsc.bitcast(d, jnp.int32)
    i = jnp.int32(0x5F3759DF) - (i >> 1)
    r = plsc.bitcast(i, jnp.float32)
    for _ in range(4):
        r = r * (1.5 - 0.5 * d * r * r)
    return r


def _sc_body(xall, src_h, dst_h, ew_h, yflat, wout,
             src_t, dst_t, wnorm_t, dinv_t, wsum_t, rows, zacc,
             zbuf40, gidx, didx, idxm, acc, dmerged, wmerged):
    c = lax.axis_index("c")
    s = lax.axis_index("s")
    zero16 = jnp.zeros((16,), jnp.float32)
    iota16 = lax.iota(jnp.int32, 16)

    def _z128(i, _):
        for u in range(HD // 16):
            zacc[i, pl.ds(16 * u, 16)] = zero16
        return 0
    lax.fori_loop(0, 128, _z128, 0)

    def _z40(i, _):
        zbuf40[i, :] = zero16
        return 0
    lax.fori_loop(0, NPAD // (16 * NSUB), _z40, 0)

    def _z640(i, _):
        dinv_t[i, :] = zero16
        wsum_t[i, :] = zero16
        return 0
    lax.fori_loop(0, NPAD // 16, _z640, 0)

    e0 = s * EPT
    pltpu.sync_copy(src_h.at[pl.ds(e0, EPT)], src_t)
    pltpu.sync_copy(dst_h.at[pl.ds(e0, EPT)], dst_t)
    pltpu.sync_copy(ew_h.at[pl.ds(e0, EPT)], wnorm_t)

    def _deg(g, _):
        sl = pl.ds(16 * g, 16)
        dv = dst_t[sl]
        plsc.addupdate_scatter(dinv_t, [dv >> 4, dv & 15], wnorm_t[sl])
        return 0
    lax.fori_loop(0, EPT // 16, _deg, 0)

    nstripe = NPAD // (16 * NSUB)

    def _merge(part_ref, out_spmem):
        pltpu.sync_copy(zbuf40, out_spmem.at[pl.ds(s * nstripe, nstripe)])
        plsc.subcore_barrier()
        for cm in range((NPAD // 16) // 128):
            for g in range(8):
                idxm[pl.ds(16 * g, 16)] = iota16 + (128 * cm + 16 * g)
            pltpu.sync_copy(part_ref.at[pl.ds(128 * cm, 128)],
                            out_spmem.at[idxm], add=True)
        plsc.subcore_barrier()

    _merge(dinv_t, dmerged)

    pltpu.sync_copy(dmerged, dinv_t)
    def _dinv(i, _):
        d = dinv_t[i, :]
        dinv_t[i, :] = jnp.where(d > 0.0, _rsqrt16(d), 0.0)
        return 0
    lax.fori_loop(0, NPAD // 16, _dinv, 0)

    def _norm(g, _):
        sl = pl.ds(16 * g, 16)
        sv = src_t[sl]
        dv = dst_t[sl]
        nm = (plsc.load_gather(dinv_t, [sv >> 4, sv & 15]) * wnorm_t[sl]
              * plsc.load_gather(dinv_t, [dv >> 4, dv & 15]))
        wnorm_t[sl] = nm
        plsc.addupdate_scatter(wsum_t, [sv >> 4, sv & 15], nm)
        return 0
    lax.fori_loop(0, EPT // 16, _norm, 0)

    _merge(wsum_t, wmerged)

    @pl.when(jnp.logical_and(c == 0, s == 0))
    def _():
        pltpu.sync_copy(wmerged, wout)

    row0 = s * ROWS_MAIN

    def _zero_stripe(nrows):
        for cz in range(nrows // 128):
            pltpu.sync_copy(zacc, acc.at[pl.ds(row0 + 128 * cz, 128)])
        rem = nrows % 128
        if rem:
            pltpu.sync_copy(zacc.at[pl.ds(0, rem)],
                            acc.at[pl.ds(row0 + (nrows // 128) * 128, rem)])

    def _pass(p, _):
        t = p // NH
        half = p % NH

        @pl.when(s < NSUB - 1)
        def _():
            _zero_stripe(ROWS_MAIN)

        @pl.when(s == NSUB - 1)
        def _():
            _zero_stripe(ROWS_LAST)
        plsc.subcore_barrier()

        base = (c * T + t) * N
        gbase = NH * base + half

        def _chunk(k, _):
            ce = k * CHUNK
            for g in range(CHUNK // 16):
                sl = pl.ds(ce + 16 * g, 16)
                gidx[pl.ds(16 * g, 16)] = NH * src_t[sl] + gbase
                didx[pl.ds(16 * g, 16)] = dst_t[sl]
            pltpu.sync_copy(xall.at[gidx], rows)

            def _edge(j, _):
                vn = plsc.load_gather(wnorm_t, [jnp.full((16,), ce + j, jnp.int32)])
                for u in range(HD // 16):
                    sl2 = pl.ds(16 * u, 16)
                    rows[j, sl2] = rows[j, sl2] * vn
                return 0
            lax.fori_loop(0, CHUNK, _edge, 0)

            pltpu.sync_copy(rows, acc.at[didx], add=True)
            return 0
        lax.fori_loop(0, NCHUNKS, _chunk, 0)

        plsc.subcore_barrier()

        @pl.when(s < NSUB - 1)
        def _():
            pltpu.sync_copy(acc.at[pl.ds(row0, ROWS_MAIN)],
                            yflat.at[half, pl.ds(base + row0, ROWS_MAIN)])

        @pl.when(s == NSUB - 1)
        def _():
            pltpu.sync_copy(acc.at[pl.ds(row0, ROWS_LAST)],
                            yflat.at[half, pl.ds(base + row0, ROWS_LAST)])
        plsc.subcore_barrier()
        return 0
    lax.fori_loop(0, NH * T, _pass, 0)


def _sc_stage(xall, src, dst, ew):
    mesh = plsc.VectorSubcoreMesh(core_axis_name="c", subcore_axis_name="s")
    f = functools.partial(
        pl.kernel,
        out_type=[
            jax.ShapeDtypeStruct((NH, NCORES * T * N, HD), jnp.float32),
            jax.ShapeDtypeStruct((NPAD // 16, 16), jnp.float32),
        ],
        mesh=mesh,
        compiler_params=pltpu.CompilerParams(
            needs_layout_passes=False, use_tc_tiling_on_sc=False),
        scratch_types=[
            pltpu.VMEM((EPT,), jnp.int32),
            pltpu.VMEM((EPT,), jnp.int32),
            pltpu.VMEM((EPT,), jnp.float32),
            pltpu.VMEM((NPAD // 16, 16), jnp.float32),
            pltpu.VMEM((NPAD // 16, 16), jnp.float32),
            pltpu.VMEM((CHUNK, HD), jnp.float32),
            pltpu.VMEM((128, HD), jnp.float32),
            pltpu.VMEM((NPAD // (16 * NSUB), 16), jnp.float32),
            pltpu.VMEM((CHUNK,), jnp.int32),
            pltpu.VMEM((CHUNK,), jnp.int32),
            pltpu.VMEM((128,), jnp.int32),
            pltpu.VMEM_SHARED((N, HD), jnp.float32),
            pltpu.VMEM_SHARED((NPAD // 16, 16), jnp.float32),
            pltpu.VMEM_SHARED((NPAD // 16, 16), jnp.float32),
        ],
    )(_sc_body)
    return f(xall, src, dst, ew)


def _dense_body(y_ref, w_ref, W1_ref, b1_ref, s_ref):
    c = pl.program_id(0)
    t = pl.program_id(1)
    h = jnp.dot(y_ref[0, 0], W1_ref[pl.ds(0, HD), :],
                preferred_element_type=jnp.float32)
    for u in range(1, NH):
        h += jnp.dot(y_ref[u, 0], W1_ref[pl.ds(u * HD, HD), :],
                     preferred_element_type=jnp.float32)
    h = jnp.maximum(h + b1_ref[...], 0.0)
    sv = jnp.dot(w_ref[...], h, preferred_element_type=jnp.float32)

    @pl.when(jnp.logical_and(c == 0, t == 0))
    def _():
        s_ref[...] = jnp.zeros_like(s_ref)
    s_ref[pl.ds(c, 1), :] += sv


def _dense_stage(yflat, w2d, W1, b1):
    return pl.pallas_call(
        _dense_body,
        grid=(NCORES, T),
        in_specs=[
            pl.BlockSpec((NH, 1, N, HD), lambda c, t: (0, c * T + t, 0, 0)),
            pl.BlockSpec((1, N), lambda c, t: (0, 0)),
            pl.BlockSpec((D, D), lambda c, t: (0, 0)),
            pl.BlockSpec((1, D), lambda c, t: (0, 0)),
        ],
        out_specs=pl.BlockSpec((NCORES, D), lambda c, t: (0, 0)),
        out_shape=jax.ShapeDtypeStruct((NCORES, D), jnp.float32),
    )(yflat, w2d, W1, b1)


def _head_body(s_ref, W2_ref, b2_ref, o_ref):
    m = jnp.dot(s_ref[...] * (1.0 / (T * N)), W2_ref[...],
                preferred_element_type=jnp.float32) + b2_ref[...]
    nrm = jnp.sqrt(jnp.sum(m * m, axis=1, keepdims=True))
    z = m / jnp.maximum(nrm, 1e-12)
    a = z[0:1, :]
    b = z[1:2, :]
    dn = (((0,), (0,)), ((), ()))
    one = jnp.ones((1, 1), jnp.float32)
    saa = lax.dot_general(a, a, dn, preferred_element_type=jnp.float32) / TEMP
    sab = lax.dot_general(a, b, dn, preferred_element_type=jnp.float32) / TEMP
    aa_col = lax.dot_general(a * a, one, dn, preferred_element_type=jnp.float32)
    ab_col = lax.dot_general(a * b, one, dn, preferred_element_type=jnp.float32)
    rowsum = (jnp.sum(jnp.exp(saa), axis=1, keepdims=True)
              - jnp.exp(aa_col / TEMP)
              + jnp.sum(jnp.exp(sab), axis=1, keepdims=True))
    denom = jnp.log(rowsum)
    pos = ab_col / TEMP
    o_ref[...] = jnp.mean(denom - pos, keepdims=True)


def _head_stage(s, W2, b2):
    return pl.pallas_call(
        _head_body,
        out_shape=jax.ShapeDtypeStruct((1, 1), jnp.float32),
    )(s, W2, b2.reshape(1, D))


def kernel(view1, view2, spatial_edge_index, spatial_edge_weight, W1, b1, W2, b2):
    xall = jnp.concatenate(
        [view1.reshape(T * N, D), view2.reshape(T * N, D)],
        axis=0).reshape(NH * NCORES * T * N, HD)
    src = spatial_edge_index[0]
    dst = spatial_edge_index[1]
    yflat, wpad = _sc_stage(xall, src, dst, spatial_edge_weight)
    y = yflat.reshape(NH, NCORES * T, N, HD)
    w2d = wpad.reshape(NPAD)[:N].reshape(1, N)
    s = _dense_stage(y, w2d, W1, b1.reshape(1, D))
    out = _head_stage(s, W2, b2)
    return out[0, 0]
